# Initial kernel scaffold; baseline (speedup 1.0000x reference)
import jax
import jax.numpy as jnp
from jax.experimental import pallas as pl


def kernel(x, edge_index, edge_attr, W_in, b_in, W0, We0, as0, ad0, ae0, b0, W1, We1, as1, ad1, ae1, b1, W2, We2, as2, ad2, ae2, b2, W_out, b_out):
    raise NotImplementedError("write your pallas kernel here")


# speedup vs baseline: 1.2154x; 1.2154x over previous
import functools

import jax
import jax.numpy as jnp
from jax.experimental import pallas as pl

N = 10000
E = 160000
D_IN = 256
HID = 512
HEADS = 8
D_HEAD = 64
D_EDGE = 16
OUT_DIM = 1


def _proj_kernel(h_ref, w_ref, b_ref, o_ref):
    o_ref[...] = jnp.dot(h_ref[...], w_ref[...],
                         preferred_element_type=jnp.float32) + b_ref[...]


def _final_proj(h, W_out, b_out):
    blk = 1000
    return pl.pallas_call(
        _proj_kernel,
        grid=(N // blk,),
        in_specs=[
            pl.BlockSpec((blk, HID), lambda i: (i, 0)),
            pl.BlockSpec((HID, OUT_DIM), lambda i: (0, 0)),
            pl.BlockSpec((OUT_DIM,), lambda i: (0,)),
        ],
        out_specs=pl.BlockSpec((blk, OUT_DIM), lambda i: (i, 0)),
        out_shape=jax.ShapeDtypeStruct((N, OUT_DIM), jnp.float32),
    )(h, W_out, b_out)


def _gat_layer(h, src, dst, ae, ae_loop, W, a_src, a_dst, bias, heads, d_head, concat):
    n = h.shape[0]
    xl = (h @ W).reshape(n, heads, d_head)
    asrc = jnp.sum(xl * a_src, axis=-1)
    adst = jnp.sum(xl * a_dst, axis=-1)
    alpha = asrc[src] + adst[dst] + ae
    alpha = jax.nn.leaky_relu(alpha, negative_slope=0.2)
    ex = jnp.exp(alpha)
    alpha_l = jax.nn.leaky_relu(asrc + adst + ae_loop, negative_slope=0.2)
    exl = jnp.exp(alpha_l)
    den = exl + jax.ops.segment_sum(ex, dst, num_segments=n)
    num = exl[:, :, None] * xl + jax.ops.segment_sum(
        ex[:, :, None] * xl[src], dst, num_segments=n)
    out = num / den[:, :, None]
    if concat:
        out = out.reshape(n, heads * d_head)
    else:
        out = jnp.mean(out, axis=1)
    return out + bias


def kernel(x, edge_index, edge_attr, W_in, b_in, W0, We0, as0, ad0, ae0, b0,
           W1, We1, as1, ad1, ae1, b1, W2, We2, as2, ad2, ae2, b2, W_out, b_out):
    src, dst = edge_index[0], edge_index[1]
    mean_ea = jnp.mean(edge_attr, axis=0)

    ea_bf = edge_attr.astype(jnp.bfloat16).astype(jnp.float32)
    mean_ea_bf = mean_ea.astype(jnp.bfloat16).astype(jnp.float32)

    def edge_B(We, a_edge, heads, d_head):
        Wr = We.astype(jnp.bfloat16).astype(jnp.float32).reshape(
            D_EDGE, heads, d_head)
        return jnp.einsum('khd,hd->kh', Wr, a_edge,
                          precision=jax.lax.Precision.HIGHEST)

    B0 = edge_B(We0, ae0, HEADS, D_HEAD)
    B1 = edge_B(We1, ae1, HEADS, D_HEAD)
    B2 = edge_B(We2, ae2, 1, HID)
    hi = jax.lax.Precision.HIGHEST
    ae0_full = jnp.dot(ea_bf, B0, precision=hi)
    ae1_full = jnp.dot(ea_bf, B1, precision=hi)
    ae2_full = jnp.dot(ea_bf, B2, precision=hi)
    ae0_loop = jnp.dot(mean_ea_bf, B0, precision=hi)
    ae1_loop = jnp.dot(mean_ea_bf, B1, precision=hi)
    ae2_loop = jnp.dot(mean_ea_bf, B2, precision=hi)

    h = x @ W_in + b_in
    h = jax.nn.relu(_gat_layer(h, src, dst, ae0_full, ae0_loop,
                               W0, as0, ad0, b0, HEADS, D_HEAD, True))
    h = jax.nn.relu(_gat_layer(h, src, dst, ae1_full, ae1_loop,
                               W1, as1, ad1, b1, HEADS, D_HEAD, True))
    h = jax.nn.relu(_gat_layer(h, src, dst, ae2_full, ae2_loop,
                               W2, as2, ad2, b2, 1, HID, False))
    return _final_proj(h, W_out, b_out)


# speedup vs baseline: 21.3189x; 17.5406x over previous
import functools

import jax
import jax.numpy as jnp
from jax import lax
from jax.experimental import pallas as pl
from jax.experimental.pallas import tpu as pltpu, tpu_sc as plsc

N = 10000
E = 160000
D_IN = 256
HID = 512
HEADS = 8
D_HEAD = 64
D_EDGE = 16
OUT_DIM = 1

NC, NS, L = 2, 16, 16
NT = NC * NS
EP = 163840
ET = EP // NS
CH = 32
NCH = ET // CH
ACC_W = 144
NPT = N // NS

_mesh = plsc.VectorSubcoreMesh(core_axis_name="c", subcore_axis_name="s")


@functools.partial(
    pl.kernel,
    out_type=jax.ShapeDtypeStruct((4, N, ACC_W), jnp.float32),
    mesh=_mesh,
    compiler_params=pltpu.CompilerParams(use_tc_tiling_on_sc=False),
    scratch_types=[
        pltpu.VMEM((ET,), jnp.int32),
        pltpu.VMEM((ET,), jnp.int32),
        pltpu.VMEM((2, CH), jnp.float32),
        pltpu.VMEM((2, CH), jnp.float32),
        pltpu.VMEM((2, CH), jnp.int32),
        pltpu.VMEM((2, CH), jnp.int32),
        pltpu.VMEM((2, CH), jnp.int32),
        pltpu.VMEM((2, CH, 128), jnp.float32),
        pltpu.VMEM((2, CH, 16), jnp.float32),
        pltpu.VMEM((2, CH, 16), jnp.float32),
        pltpu.VMEM((CH, ACC_W), jnp.float32),
        pltpu.VMEM_SHARED((N, ACC_W), jnp.float32),
        pltpu.SemaphoreType.DMA,
    ],
)
def _edge_kernel(xl_ref, src_ref, dst_ref, logt_ref, ae_ref, parts_ref,
                 srcv, dstv, aeb0, aeb1, idxb, didxb, dstc, gbuf, sbuf,
                 dbuf, scaled, acc, sem):
    c = lax.axis_index("c")
    s = lax.axis_index("s")
    tbase = s * ET
    iota = lax.iota(jnp.int32, L)
    zero = jnp.zeros((L,), jnp.float32)
    den_pat0 = jnp.where(iota == 0, 1.0, 0.0)
    den_pat1 = jnp.where(iota == 1, 1.0, 0.0)

    pltpu.sync_copy(src_ref.at[pl.ds(tbase, ET)], srcv)
    pltpu.sync_copy(dst_ref.at[pl.ds(tbase, ET)], dstv)

    def one_pass(kk, _):
        p = c * 2 + kk
        shift = p * N
        for r in range(CH):
            for f in range(ACC_W // L):
                scaled[r, pl.ds(f * L, L)] = zero
        for q in range(NPT // CH):
            pltpu.sync_copy(
                scaled, acc.at[pl.ds(s * NPT + q * CH, CH)])
        rem = NPT - (NPT // CH) * CH
        if rem:
            pltpu.sync_copy(scaled.at[pl.ds(0, rem)],
                            acc.at[pl.ds(s * NPT + (NPT // CH) * CH, rem)])
        plsc.subcore_barrier()

        def issue(buf, j):
            for l in range(CH // L):
                idxb[buf, pl.ds(l * L, L)] = (
                    srcv[pl.ds(j * CH + l * L, L)] + shift)
                didxb[buf, pl.ds(l * L, L)] = (
                    dstv[pl.ds(j * CH + l * L, L)] + shift)
            pltpu.async_copy(xl_ref.at[idxb.at[buf]], gbuf.at[buf], sem)
            pltpu.async_copy(logt_ref.at[idxb.at[buf]], sbuf.at[buf], sem)
            pltpu.async_copy(logt_ref.at[didxb.at[buf]], dbuf.at[buf], sem)
            off0 = pl.multiple_of(2 * p * EP + tbase + j * CH, 8)
            off1 = pl.multiple_of((2 * p + 1) * EP + tbase + j * CH, 8)
            pltpu.async_copy(ae_ref.at[pl.ds(off0, CH)], aeb0.at[buf], sem)
            pltpu.async_copy(ae_ref.at[pl.ds(off1, CH)], aeb1.at[buf], sem)

        def wait(buf):
            pltpu.make_async_copy(xl_ref.at[idxb.at[buf]], gbuf.at[buf],
                                  sem).wait()
            pltpu.make_async_copy(logt_ref.at[idxb.at[buf]], sbuf.at[buf],
                                  sem).wait()
            pltpu.make_async_copy(logt_ref.at[didxb.at[buf]], dbuf.at[buf],
                                  sem).wait()
            pltpu.make_async_copy(ae_ref.at[pl.ds(0, CH)], aeb0.at[buf],
                                  sem).wait()
            pltpu.make_async_copy(ae_ref.at[pl.ds(0, CH)], aeb1.at[buf],
                                  sem).wait()

        def compute(buf, j):
            for g in range(CH // L):
                a0 = zero
                a1 = zero
                for l in range(L):
                    e = g * L + l
                    srow = sbuf[buf, e, pl.ds(0, L)]
                    drow = dbuf[buf, e, pl.ds(0, L)]
                    lane = (iota == l)
                    a0 = jnp.where(lane, srow[0] + drow[2], a0)
                    a1 = jnp.where(lane, srow[1] + drow[3], a1)
                a0 = a0 + aeb0[buf, pl.ds(g * L, L)]
                a1 = a1 + aeb1[buf, pl.ds(g * L, L)]
                a0 = jnp.where(a0 >= 0.0, a0, 0.2 * a0)
                a1 = jnp.where(a1 >= 0.0, a1, 0.2 * a1)
                ex0 = jnp.exp(a0)
                ex1 = jnp.exp(a1)
                for l in range(L):
                    e = g * L + l
                    w0 = ex0[l]
                    w1 = ex1[l]
                    for f in range(4):
                        scaled[e, pl.ds(f * L, L)] = (
                            gbuf[buf, e, pl.ds(f * L, L)] * w0)
                    for f in range(4, 8):
                        scaled[e, pl.ds(f * L, L)] = (
                            gbuf[buf, e, pl.ds(f * L, L)] * w1)
                    scaled[e, pl.ds(128, L)] = (den_pat0 * w0
                                                + den_pat1 * w1)
            for l in range(CH // L):
                dstc[buf, pl.ds(l * L, L)] = dstv[pl.ds(j * CH + l * L, L)]
            pltpu.sync_copy(scaled, acc.at[dstc.at[buf]], add=True)

        issue(0, 0)
        issue(1, 1)

        def chunk_pair(j2, _):
            ja = 2 * j2
            wait(0)
            compute(0, ja)

            @pl.when(ja + 2 < NCH)
            def _():
                issue(0, ja + 2)

            wait(1)
            compute(1, ja + 1)

            @pl.when(ja + 3 < NCH)
            def _():
                issue(1, ja + 3)

            return 0

        lax.fori_loop(0, NCH // 2, chunk_pair, 0)
        plsc.subcore_barrier()
        pltpu.sync_copy(acc.at[pl.ds(s * NPT, NPT)],
                        parts_ref.at[p, pl.ds(s * NPT, NPT)])
        plsc.subcore_barrier()
        return 0

    lax.fori_loop(0, 2, one_pass, 0)


def _proj_kernel(h_ref, w_ref, b_ref, o_ref):
    o_ref[...] = jnp.dot(h_ref[...], w_ref[...],
                         preferred_element_type=jnp.float32) + b_ref[...]


def _final_proj(h, W_out, b_out):
    blk = 1000
    return pl.pallas_call(
        _proj_kernel,
        grid=(N // blk,),
        in_specs=[
            pl.BlockSpec((blk, HID), lambda i: (i, 0)),
            pl.BlockSpec((HID, OUT_DIM), lambda i: (0, 0)),
            pl.BlockSpec((OUT_DIM,), lambda i: (0,)),
        ],
        out_specs=pl.BlockSpec((blk, OUT_DIM), lambda i: (i, 0)),
        out_shape=jax.ShapeDtypeStruct((N, OUT_DIM), jnp.float32),
    )(h, W_out, b_out)


def _pack_tabs(asrc, adst):
    rows = []
    for p in range(4):
        r = jnp.stack([asrc[:, 2 * p], asrc[:, 2 * p + 1],
                       adst[:, 2 * p], adst[:, 2 * p + 1]], axis=1)
        rows.append(jnp.pad(r, ((0, 0), (0, 12))))
    return jnp.concatenate(rows, axis=0)


def _gat_layer_sc(h, src_p, dst_p, ae_full, ae_loop, W, a_src, a_dst, bias,
                  heads, d_head):
    xl = h @ W
    xlr = xl.reshape(N, heads, d_head)
    asrc = jnp.sum(xlr * a_src, axis=-1)
    adst = jnp.sum(xlr * a_dst, axis=-1)
    if heads == 1:
        asrc8 = jnp.repeat(asrc, 8, axis=1)
        adst8 = jnp.repeat(adst, 8, axis=1)
        ae8 = jnp.repeat(ae_full, 8, axis=1)
        exl = jnp.exp(jnp.where(
            (asrc + adst + ae_loop) >= 0.0,
            asrc + adst + ae_loop, 0.2 * (asrc + adst + ae_loop)))
    else:
        asrc8, adst8, ae8 = asrc, adst, ae_full
        al = asrc + adst + ae_loop
        exl = jnp.exp(jnp.where(al >= 0.0, al, 0.2 * al))

    tabs = _pack_tabs(asrc8, adst8)
    ae_pad = jnp.full((8, EP), -100.0, jnp.float32)
    ae_pad = ae_pad.at[:, :E].set(ae8.T)
    xl_flat = jnp.transpose(xl.reshape(N, 4, 128), (1, 0, 2)).reshape(
        4 * N, 128)

    parts = _edge_kernel(xl_flat, src_p, dst_p, tabs, ae_pad.reshape(-1))

    num = jnp.transpose(parts[:, :, :128], (1, 0, 2)).reshape(N, 8, 64)
    den = jnp.transpose(parts[:, :, 128:130], (1, 0, 2)).reshape(N, 8)
    if heads == 1:
        exl8 = jnp.repeat(exl, 8, axis=1)
    else:
        exl8 = exl
    out = ((num + exl8[:, :, None] * xl.reshape(N, 8, 64))
           / (den + exl8)[:, :, None])
    return out.reshape(N, HID) + bias


def kernel(x, edge_index, edge_attr, W_in, b_in, W0, We0, as0, ad0, ae0, b0,
           W1, We1, as1, ad1, ae1, b1, W2, We2, as2, ad2, ae2, b2, W_out,
           b_out):
    src, dst = edge_index[0], edge_index[1]
    pad_ids = (jnp.arange(EP - E, dtype=jnp.int32) * 37) % N
    src_p = jnp.concatenate([src, pad_ids])
    dst_p = jnp.concatenate([dst, pad_ids])

    mean_ea = jnp.mean(edge_attr, axis=0)
    ea_bf = edge_attr.astype(jnp.bfloat16).astype(jnp.float32)
    mean_ea_bf = mean_ea.astype(jnp.bfloat16).astype(jnp.float32)

    def edge_B(We, a_edge, heads, d_head):
        Wr = We.astype(jnp.bfloat16).astype(jnp.float32).reshape(
            D_EDGE, heads, d_head)
        return jnp.einsum('khd,hd->kh', Wr, a_edge,
                          precision=lax.Precision.HIGHEST)

    hi = lax.Precision.HIGHEST
    B0 = edge_B(We0, ae0, HEADS, D_HEAD)
    B1 = edge_B(We1, ae1, HEADS, D_HEAD)
    B2 = edge_B(We2, ae2, 1, HID)
    aef0 = jnp.dot(ea_bf, B0, precision=hi)
    aef1 = jnp.dot(ea_bf, B1, precision=hi)
    aef2 = jnp.dot(ea_bf, B2, precision=hi)
    ael0 = jnp.dot(mean_ea_bf, B0, precision=hi)
    ael1 = jnp.dot(mean_ea_bf, B1, precision=hi)
    ael2 = jnp.dot(mean_ea_bf, B2, precision=hi)

    h = x @ W_in + b_in
    h = jax.nn.relu(_gat_layer_sc(h, src_p, dst_p, aef0, ael0, W0,
                                  as0, ad0, b0, HEADS, D_HEAD))
    h = jax.nn.relu(_gat_layer_sc(h, src_p, dst_p, aef1, ael1, W1,
                                  as1, ad1, b1, HEADS, D_HEAD))
    h = jax.nn.relu(_gat_layer_sc(h, src_p, dst_p, aef2, ael2, W2,
                                  as2, ad2, b2, 1, HID))
    return _final_proj(h, W_out, b_out)
